# baseline (speedup 1.0000x reference)
import functools

import jax
import jax.numpy as jnp
from jax import lax
from jax.experimental import pallas as pl
from jax.experimental.pallas import tpu as pltpu
from jax.experimental.pallas import tpu_sc as plsc

N_CLS = 1000
N_CTX = 4
CTX_DIM = 512
CTX_LEN = 77
SUFFIX_LEN = CTX_LEN - 1 - N_CTX

_NC = 2
_NS = 16
_NW = _NC * _NS

_BASE = N_CLS // _NW
_REM = N_CLS % _NW


def _prompt_body(prefix_hbm, ctx_hbm, suffix_hbm, out_hbm, ctx_v, sem_s, sem_p, sem_c):
    wid = lax.axis_index("s") * _NC + lax.axis_index("c")
    lo = wid * _BASE + jnp.minimum(wid, _REM)
    cnt = _BASE + jnp.where(wid < _REM, 1, 0)

    pltpu.sync_copy(ctx_hbm, ctx_v)

    def body(j, carry):
        i = lo + j
        c_s = pltpu.async_copy(suffix_hbm.at[i], out_hbm.at[i, pl.ds(N_CTX + 1, SUFFIX_LEN)], sem_s)
        c_p = pltpu.async_copy(prefix_hbm.at[i], out_hbm.at[i, pl.ds(0, 1)], sem_p)
        c_c = pltpu.async_copy(ctx_v, out_hbm.at[i, pl.ds(1, N_CTX)], sem_c)
        c_s.wait()
        c_p.wait()
        c_c.wait()
        return carry

    lax.fori_loop(0, cnt, body, 0)


def kernel(prefixs, ctx, suffixs):
    mesh = plsc.VectorSubcoreMesh(core_axis_name="c", subcore_axis_name="s")
    run = pl.kernel(
        _prompt_body,
        out_type=jax.ShapeDtypeStruct((N_CLS, CTX_LEN, CTX_DIM), jnp.float32),
        mesh=mesh,
        scratch_types=[
            pltpu.VMEM((N_CTX, CTX_DIM), jnp.float32),
            pltpu.SemaphoreType.DMA,
            pltpu.SemaphoreType.DMA,
            pltpu.SemaphoreType.DMA,
        ],
        compiler_params=pltpu.CompilerParams(use_tc_tiling_on_sc=False),
    )
    return run(prefixs, ctx, suffixs)


# speedup vs baseline: 22.2396x; 22.2396x over previous
import functools

import jax
import jax.numpy as jnp
from jax.experimental import pallas as pl
from jax.experimental.pallas import tpu as pltpu

N_CLS = 1000
N_CTX = 4
CTX_DIM = 512
CTX_LEN = 77
SUFFIX_LEN = CTX_LEN - 1 - N_CTX

CB = 8


def _body(prefix_ref, ctx_ref, suffix_ref, out_ref):
    out_ref[:, 0:1, :] = prefix_ref[...]
    out_ref[:, 1:1 + N_CTX, :] = jnp.broadcast_to(
        ctx_ref[...][None], (CB, N_CTX, CTX_DIM))
    out_ref[:, 1 + N_CTX:, :] = suffix_ref[...]


def kernel(prefixs, ctx, suffixs):
    grid = (N_CLS // CB,)
    return pl.pallas_call(
        _body,
        grid=grid,
        in_specs=[
            pl.BlockSpec((CB, 1, CTX_DIM), lambda i: (i, 0, 0)),
            pl.BlockSpec((N_CTX, CTX_DIM), lambda i: (0, 0)),
            pl.BlockSpec((CB, SUFFIX_LEN, CTX_DIM), lambda i: (i, 0, 0)),
        ],
        out_specs=pl.BlockSpec((CB, CTX_LEN, CTX_DIM), lambda i: (i, 0, 0)),
        out_shape=jax.ShapeDtypeStruct((N_CLS, CTX_LEN, CTX_DIM), jnp.float32),
        compiler_params=pltpu.CompilerParams(
            dimension_semantics=("arbitrary",),
        ),
    )(prefixs, ctx, suffixs)


# speedup vs baseline: 26.4190x; 1.1879x over previous
import functools

import jax
import jax.numpy as jnp
from jax.experimental import pallas as pl
from jax.experimental.pallas import tpu as pltpu

N_CLS = 1000
N_CTX = 4
CTX_DIM = 512
CTX_LEN = 77
SUFFIX_LEN = CTX_LEN - 1 - N_CTX

CB = 40


def _body(prefix_ref, ctx_ref, suffix_ref, out_ref):
    out_ref[:, 0:1, :] = prefix_ref[...]
    out_ref[:, 1:1 + N_CTX, :] = jnp.broadcast_to(
        ctx_ref[...][None], (CB, N_CTX, CTX_DIM))
    out_ref[:, 1 + N_CTX:, :] = suffix_ref[...]


def kernel(prefixs, ctx, suffixs):
    grid = (N_CLS // CB,)
    return pl.pallas_call(
        _body,
        grid=grid,
        in_specs=[
            pl.BlockSpec((CB, 1, CTX_DIM), lambda i: (i, 0, 0)),
            pl.BlockSpec((N_CTX, CTX_DIM), lambda i: (0, 0)),
            pl.BlockSpec((CB, SUFFIX_LEN, CTX_DIM), lambda i: (i, 0, 0)),
        ],
        out_specs=pl.BlockSpec((CB, CTX_LEN, CTX_DIM), lambda i: (i, 0, 0)),
        out_shape=jax.ShapeDtypeStruct((N_CLS, CTX_LEN, CTX_DIM), jnp.float32),
        compiler_params=pltpu.CompilerParams(
            dimension_semantics=("arbitrary",),
        ),
    )(prefixs, ctx, suffixs)


# speedup vs baseline: 26.4587x; 1.0015x over previous
import functools

import jax
import jax.numpy as jnp
from jax.experimental import pallas as pl
from jax.experimental.pallas import tpu as pltpu

N_CLS = 1000
N_CTX = 4
CTX_DIM = 512
CTX_LEN = 77
SUFFIX_LEN = CTX_LEN - 1 - N_CTX

CB = 50


def _body(prefix_ref, ctx_ref, suffix_ref, out_ref):
    out_ref[:, 0:1, :] = prefix_ref[...]
    out_ref[:, 1:1 + N_CTX, :] = jnp.broadcast_to(
        ctx_ref[...][None], (CB, N_CTX, CTX_DIM))
    out_ref[:, 1 + N_CTX:, :] = suffix_ref[...]


def kernel(prefixs, ctx, suffixs):
    grid = (N_CLS // CB,)
    return pl.pallas_call(
        _body,
        grid=grid,
        in_specs=[
            pl.BlockSpec((CB, 1, CTX_DIM), lambda i: (i, 0, 0)),
            pl.BlockSpec((N_CTX, CTX_DIM), lambda i: (0, 0)),
            pl.BlockSpec((CB, SUFFIX_LEN, CTX_DIM), lambda i: (i, 0, 0)),
        ],
        out_specs=pl.BlockSpec((CB, CTX_LEN, CTX_DIM), lambda i: (i, 0, 0)),
        out_shape=jax.ShapeDtypeStruct((N_CLS, CTX_LEN, CTX_DIM), jnp.float32),
        compiler_params=pltpu.CompilerParams(
            dimension_semantics=("arbitrary",),
        ),
    )(prefixs, ctx, suffixs)
